# baseline (speedup 1.0000x reference)
import functools

import jax
import jax.numpy as jnp
from jax import lax
from jax.experimental import pallas as pl
from jax.experimental.pallas import tpu as pltpu
from jax.experimental.pallas import tpu_sc as plsc

_B, _S, _D = 2, 2048, 1024
_T = _B * _S
_THRESHOLD = 128
_XB = 256
_TPAD = _T + 2 * _XB
_NBLK = _TPAD // _XB

_NW = 32
_GCH = 64
_TPW = _T // _NW


def _sc_gather(idx, table):
  t, d = idx.shape[0], table.shape[1]
  b_per_w = t // _NW
  n_ch = b_per_w // _GCH
  mesh = plsc.VectorSubcoreMesh(core_axis_name="c", subcore_axis_name="s")

  @functools.partial(
      pl.kernel,
      mesh=mesh,
      out_type=jax.ShapeDtypeStruct((t, d), jnp.float32),
      scratch_types=[
          pltpu.VMEM((_GCH,), jnp.int32),
          pltpu.VMEM((_GCH, d), jnp.float32),
          pltpu.SemaphoreType.DMA,
      ],
  )
  def k(idx_hbm, table_hbm, out_hbm, idx_v, rows_v, sem):
    wid = lax.axis_index("s") * 2 + lax.axis_index("c")
    base = wid * b_per_w
    for c in range(n_ch):
      off = base + c * _GCH
      pltpu.sync_copy(idx_hbm.at[pl.ds(off, _GCH)], idx_v)
      pltpu.async_copy(table_hbm.at[idx_v], rows_v, sem).wait()
      pltpu.sync_copy(rows_v, out_hbm.at[pl.ds(off, _GCH)])

  return k(idx, table)


def _sc_scatter2(pos0, pos1, x0, x1):
  d = x0.shape[1]
  ch = 32
  n_ch = _TPW // ch
  mesh = plsc.VectorSubcoreMesh(core_axis_name="c", subcore_axis_name="s")

  @functools.partial(
      pl.kernel,
      mesh=mesh,
      out_type=jax.ShapeDtypeStruct((2 * _TPAD, d), jnp.float32),
      scratch_types=[
          pltpu.VMEM((ch,), jnp.int32),
          pltpu.VMEM((ch,), jnp.int32),
          pltpu.VMEM((ch, d), jnp.float32),
          pltpu.VMEM((ch, d), jnp.float32),
          pltpu.SemaphoreType.DMA,
          pltpu.SemaphoreType.DMA,
      ],
  )
  def k(p0_h, p1_h, x0_h, x1_h, out_h, i0_v, i1_v, r0_v, r1_v, s0, s1):
    wid = lax.axis_index("s") * 2 + lax.axis_index("c")
    base = wid * _TPW
    idxs = (i0_v, i1_v)
    rows = (r0_v, r1_v)
    sems = (s0, s1)
    prev = [None, None]
    step = 0
    for part in range(2):
      pos_h = p0_h if part == 0 else p1_h
      x_h = x0_h if part == 0 else x1_h
      for c in range(n_ch):
        b = step % 2
        if prev[b] is not None:
          prev[b].wait()
        off = base + c * ch
        pltpu.sync_copy(pos_h.at[pl.ds(off, ch)], idxs[b])
        pltpu.sync_copy(x_h.at[pl.ds(off, ch)], rows[b])
        prev[b] = pltpu.async_copy(rows[b], out_h.at[idxs[b]], sems[b])
        step += 1
    prev[0].wait()
    prev[1].wait()

  return k(pos0, pos1, x0, x1)


def _sc_combine(psem, psyn_off, s0, s2, alllog, lenlog):
  mesh = plsc.VectorSubcoreMesh(core_axis_name="c", subcore_axis_name="s")

  @functools.partial(
      pl.kernel,
      mesh=mesh,
      out_type=jax.ShapeDtypeStruct((2 * _T,), jnp.float32),
      compiler_params=pltpu.CompilerParams(needs_layout_passes=False),
      scratch_types=[
          pltpu.VMEM((4 * _TPAD,), jnp.float32),
          pltpu.VMEM((_TPW,), jnp.int32),
          pltpu.VMEM((_TPW,), jnp.int32),
          pltpu.VMEM((_TPW,), jnp.float32),
          pltpu.VMEM((_TPW,), jnp.float32),
          pltpu.VMEM((2 * _TPW,), jnp.float32),
          pltpu.VMEM((2 * _TPW,), jnp.float32),
      ],
  )
  def k(psem_h, psyn_h, s0_h, s2_h, alll_h, lenl_h, out_h,
        all_v, pm_v, ps_v, s0_v, s2_v, len_v, out_v):
    wid = lax.axis_index("s") * 2 + lax.axis_index("c")
    base = wid * _TPW
    pltpu.sync_copy(alll_h, all_v)
    pltpu.sync_copy(psem_h.at[pl.ds(base, _TPW)], pm_v)
    pltpu.sync_copy(psyn_h.at[pl.ds(base, _TPW)], ps_v)
    pltpu.sync_copy(s0_h.at[pl.ds(base, _TPW)], s0_v)
    pltpu.sync_copy(s2_h.at[pl.ds(base, _TPW)], s2_v)
    pltpu.sync_copy(lenl_h.at[pl.ds(2 * base, 2 * _TPW)], len_v)
    for c in range(_TPW // 16):
      sl = pl.ds(c * 16, 16)
      pi2 = ps_v[sl] * 2
      qi2 = pm_v[sl] * 2
      ss = s0_v[sl]
      ms = s2_v[sl]
      tok2 = (lax.iota(jnp.int32, 16) + c * 16) * 2
      a0 = plsc.load_gather(all_v, [pi2])
      a1 = plsc.load_gather(all_v, [pi2 + 1])
      b0 = plsc.load_gather(all_v, [qi2])
      b1 = plsc.load_gather(all_v, [qi2 + 1])
      l0 = plsc.load_gather(len_v, [tok2])
      l1 = plsc.load_gather(len_v, [tok2 + 1])
      plsc.store_scatter(out_v, [tok2], ss * a0 + ms * b0 + l0)
      plsc.store_scatter(out_v, [tok2 + 1], ss * a1 + ms * b1 + l1)
    pltpu.sync_copy(out_v, out_h.at[pl.ds(2 * base, 2 * _TPW)])

  return k(psem, psyn_off, s0, s2, alllog, lenlog)


_RTB = 512


def _mm1_router_body(short_ref, hs_ref, g1w_ref, w_ref, b_ref,
                     t1_ref, idx_ref, scl_ref):
  hs = hs_ref[...]
  t1_ref[...] = jnp.dot(hs, g1w_ref[...], preferred_element_type=jnp.float32)
  rl = jnp.dot(hs, w_ref[...], preferred_element_type=jnp.float32)
  rl = rl + b_ref[...]
  short = short_ref[0, 0, 0] != 0
  neg = jnp.float32(-1e9)
  col = lax.broadcasted_iota(jnp.int32, rl.shape, 1)
  rl = jnp.where(jnp.logical_and(col == 4, short), neg, rl)
  rl = jnp.where(jnp.logical_and(col == 3, jnp.logical_not(short)), neg, rl)
  m = jnp.max(rl, axis=-1, keepdims=True)
  e = jnp.exp(rl - m)
  p = e / jnp.sum(e, axis=-1, keepdims=True)
  syn_p = jnp.max(p[:, 0:3], axis=-1, keepdims=True)
  syn_i = jnp.argmax(p[:, 0:3], axis=-1, keepdims=True)
  len_p = jnp.max(p[:, 3:5], axis=-1, keepdims=True)
  sem_p = jnp.max(p[:, 5:8], axis=-1, keepdims=True)
  sem_i = jnp.argmax(p[:, 5:8], axis=-1, keepdims=True)
  tot = syn_p + len_p + sem_p
  scl_ref[...] = jnp.concatenate([syn_p, len_p, sem_p, tot], axis=-1) / tot
  idx_ref[...] = jnp.concatenate([syn_i, sem_i], axis=-1).astype(jnp.int32)


def _mm1_router(hs2d, gcn1_W, router_W, router_b, is_short):
  return pl.pallas_call(
      _mm1_router_body,
      grid=(_T // _RTB,),
      in_specs=[
          pl.BlockSpec((1, 1, 1), lambda i: (i * _RTB // _S, 0, 0)),
          pl.BlockSpec((_RTB, _D), lambda i: (i, 0)),
          pl.BlockSpec((_D, _D), lambda i: (0, 0)),
          pl.BlockSpec((_D, 8), lambda i: (0, 0)),
          pl.BlockSpec((1, 8), lambda i: (0, 0)),
      ],
      out_specs=[
          pl.BlockSpec((_RTB, _D), lambda i: (i, 0)),
          pl.BlockSpec((_RTB, 2), lambda i: (i, 0)),
          pl.BlockSpec((_RTB, 4), lambda i: (i, 0)),
      ],
      out_shape=[
          jax.ShapeDtypeStruct((_T, _D), jnp.float32),
          jax.ShapeDtypeStruct((_T, 2), jnp.int32),
          jax.ShapeDtypeStruct((_T, 4), jnp.float32),
      ],
  )(is_short, hs2d, gcn1_W, router_W, router_b.reshape(1, 8))


_PR, _PC = _T // 128, 128


def _pos_body(syn_ref, sem_ref, psyn_ref, psem_ref, offs_ref):
  r128 = lax.broadcasted_iota(jnp.int32, (_PC, _PC), 0)
  c128 = lax.broadcasted_iota(jnp.int32, (_PC, _PC), 1)
  u_inc = (r128 <= c128).astype(jnp.float32)
  r32 = lax.broadcasted_iota(jnp.int32, (_PR, _PR), 0)
  c32 = lax.broadcasted_iota(jnp.int32, (_PR, _PR), 1)
  l_exc = (c32 < r32).astype(jnp.float32)

  def group(e2d):
    pos = jnp.zeros((_PR, _PC), jnp.float32)
    off = jnp.float32(0.0)
    offs = []
    for k in range(3):
      m = (e2d == k).astype(jnp.float32)
      c1 = jnp.dot(m, u_inc, preferred_element_type=jnp.float32)
      rt = jnp.sum(m, axis=1, keepdims=True)
      r0 = jnp.dot(l_exc, rt, preferred_element_type=jnp.float32)
      rank = r0 + c1 - m
      pos = jnp.where(e2d == k, off + rank, pos)
      offs.append(off)
      cnt = jnp.sum(m)
      off = off + jnp.floor((cnt + 255.0) * (1.0 / 256.0)) * 256.0
    offs.append(off)
    row = jnp.concatenate([o.reshape(1, 1) for o in offs], axis=1)
    return pos.astype(jnp.int32), row

  psyn, so = group(syn_ref[...])
  psem, mo = group(sem_ref[...])
  psyn_ref[...] = psyn
  psem_ref[...] = psem
  offs_ref[...] = jnp.concatenate([so, mo], axis=0).astype(jnp.int32)


def _pos_kernel(syn2d, sem2d):
  return pl.pallas_call(
      _pos_body,
      grid=(1,),
      in_specs=[
          pl.BlockSpec((_PR, _PC), lambda i: (0, 0)),
          pl.BlockSpec((_PR, _PC), lambda i: (0, 0)),
      ],
      out_specs=[
          pl.BlockSpec((_PR, _PC), lambda i: (0, 0)),
          pl.BlockSpec((_PR, _PC), lambda i: (0, 0)),
          pl.BlockSpec((2, 4), lambda i: (0, 0)),
      ],
      out_shape=[
          jax.ShapeDtypeStruct((_PR, _PC), jnp.int32),
          jax.ShapeDtypeStruct((_PR, _PC), jnp.int32),
          jax.ShapeDtypeStruct((2, 4), jnp.int32),
      ],
  )(syn2d, sem2d)


_MMB = 512


def _mm_body(x_ref, w_ref, o_ref):
  o_ref[...] = jnp.dot(x_ref[...], w_ref[...],
                       preferred_element_type=jnp.float32)


def _mm(x, w):
  return pl.pallas_call(
      _mm_body,
      grid=(_T // _MMB,),
      in_specs=[
          pl.BlockSpec((_MMB, _D), lambda i: (i, 0)),
          pl.BlockSpec((_D, _D), lambda i: (0, 0)),
      ],
      out_specs=pl.BlockSpec((_MMB, _D), lambda i: (i, 0)),
      out_shape=jax.ShapeDtypeStruct((_T, _D), jnp.float32),
  )(x, w)


_AMB = 256


def _adj_body(adj_ref, sup_ref, o_ref):
  a = adj_ref[0]
  deg = jnp.maximum(jnp.sum(a, axis=-1, keepdims=True), 1e-9)
  acc = jnp.dot(a, sup_ref[0], preferred_element_type=jnp.float32)
  o_ref[...] = jnp.maximum(acc / deg, 0.0)[None]


def _adj_ln_body(adj_ref, sup_ref, hs_ref, g_ref, b_ref, o_ref):
  a = adj_ref[0]
  deg = jnp.maximum(jnp.sum(a, axis=-1, keepdims=True), 1e-9)
  acc = jnp.dot(a, sup_ref[0], preferred_element_type=jnp.float32)
  x = jnp.maximum(acc / deg, 0.0) + hs_ref[0]
  mu = jnp.mean(x, axis=-1, keepdims=True)
  xc = x - mu
  var = jnp.mean(xc * xc, axis=-1, keepdims=True)
  y = xc * lax.rsqrt(var + 1e-5) * g_ref[...] + b_ref[...]
  o_ref[...] = y[None]


def _adj_mm(adj, sup3d):
  return pl.pallas_call(
      _adj_body,
      grid=(_B, _S // _AMB),
      in_specs=[
          pl.BlockSpec((1, _AMB, _S), lambda b, i: (b, i, 0)),
          pl.BlockSpec((1, _S, _D), lambda b, i: (b, 0, 0)),
      ],
      out_specs=pl.BlockSpec((1, _AMB, _D), lambda b, i: (b, i, 0)),
      out_shape=jax.ShapeDtypeStruct((_B, _S, _D), jnp.float32),
  )(adj, sup3d)


def _adj_mm_ln(adj, sup3d, hs3d, ln_g, ln_b):
  return pl.pallas_call(
      _adj_ln_body,
      grid=(_B, _S // _AMB),
      in_specs=[
          pl.BlockSpec((1, _AMB, _S), lambda b, i: (b, i, 0)),
          pl.BlockSpec((1, _S, _D), lambda b, i: (b, 0, 0)),
          pl.BlockSpec((1, _AMB, _D), lambda b, i: (b, i, 0)),
          pl.BlockSpec((1, _D), lambda b, i: (0, 0)),
          pl.BlockSpec((1, _D), lambda b, i: (0, 0)),
      ],
      out_specs=pl.BlockSpec((1, _AMB, _D), lambda b, i: (b, i, 0)),
      out_shape=jax.ShapeDtypeStruct((_B, _S, _D), jnp.float32),
  )(adj, sup3d, hs3d, ln_g.reshape(1, _D), ln_b.reshape(1, _D))


def _gelu(x):
  return 0.5 * x * (1.0 + lax.erf(x * 0.7071067811865476))


def _group_body(be_ref, x_ref, w_ref, b_ref, cls_ref, o_ref):
  del be_ref
  eo = _gelu(jnp.dot(x_ref[...], w_ref[0], preferred_element_type=jnp.float32)
             + b_ref[0])
  o_ref[...] = jnp.dot(eo, cls_ref[...], preferred_element_type=jnp.float32)


def _group_mm(be, xs, wk, bk, cls_W):
  rows = xs.shape[0]
  ne = wk.shape[0]
  grid_spec = pltpu.PrefetchScalarGridSpec(
      num_scalar_prefetch=1,
      grid=(rows // _XB,),
      in_specs=[
          pl.BlockSpec((_XB, _D), lambda i, be: (i, 0)),
          pl.BlockSpec((1, _D, _D), lambda i, be: (be[i], 0, 0)),
          pl.BlockSpec((1, 1, _D), lambda i, be: (be[i], 0, 0)),
          pl.BlockSpec((_D, 2), lambda i, be: (0, 0)),
      ],
      out_specs=pl.BlockSpec((_XB, 2), lambda i, be: (i, 0)),
  )
  return pl.pallas_call(
      _group_body,
      grid_spec=grid_spec,
      out_shape=jax.ShapeDtypeStruct((rows, 2), jnp.float32),
  )(be, xs, wk, bk.reshape(ne, 1, _D), cls_W)


def _len_body(sel_ref, scl_ref, hs_ref, w_ref, b_ref, cls_ref, clsb_ref,
              o_ref):
  del sel_ref
  lo = _gelu(jnp.dot(hs_ref[...], w_ref[0], preferred_element_type=jnp.float32)
             + b_ref[0])
  lg = jnp.dot(lo, cls_ref[...], preferred_element_type=jnp.float32)
  o_ref[...] = lg * scl_ref[...] + clsb_ref[...]


def _len_mm(len_sel, scl1, hs2d, len_W2, len_b2, cls_W, cls_b):
  grid_spec = pltpu.PrefetchScalarGridSpec(
      num_scalar_prefetch=1,
      grid=(_T // _XB,),
      in_specs=[
          pl.BlockSpec((_XB, 1), lambda i, sel: (i, 0)),
          pl.BlockSpec((_XB, _D), lambda i, sel: (i, 0)),
          pl.BlockSpec((1, _D, _D), lambda i, sel: (sel[i * _XB // _S], 0, 0)),
          pl.BlockSpec((1, 1, _D), lambda i, sel: (sel[i * _XB // _S], 0, 0)),
          pl.BlockSpec((_D, 2), lambda i, sel: (0, 0)),
          pl.BlockSpec((1, 2), lambda i, sel: (0, 0)),
      ],
      out_specs=pl.BlockSpec((_XB, 2), lambda i, sel: (i, 0)),
  )
  return pl.pallas_call(
      _len_body,
      grid_spec=grid_spec,
      out_shape=jax.ShapeDtypeStruct((_T, 2), jnp.float32),
  )(len_sel, scl1, hs2d, len_W2, len_b2.reshape(2, 1, _D), cls_W,
    cls_b.reshape(1, 2))


def kernel(input_ids, attention_mask, seq_lengths, adj_matrix, emb, router_W,
           router_b, gcn1_W, gcn2_W, ln_g, ln_b, syn_W, syn_b, lenS_W, lenS_b,
           lenL_W, lenL_b, sem_W, sem_b, cls_W, cls_b):
  del attention_mask
  ids = input_ids.reshape(_T).astype(jnp.int32)
  hs2d = _sc_gather(ids, emb)
  hs3d = hs2d.reshape(_B, _S, _D)

  is_short = (seq_lengths <= _THRESHOLD)
  t1, idx, scl = _mm1_router(hs2d, gcn1_W, router_W, router_b,
                             is_short.astype(jnp.int32).reshape(_B, 1, 1))

  psyn2d, psem2d, offs = _pos_kernel(idx[:, 0].reshape(_PR, _PC),
                                     idx[:, 1].reshape(_PR, _PC))
  psyn = psyn2d.reshape(_T)
  psem = psem2d.reshape(_T)
  blk = jnp.arange(_NBLK, dtype=jnp.int32)[:, None] * _XB
  be_syn = jnp.sum((blk >= offs[0, 1:3][None, :]).astype(jnp.int32), axis=1)
  be_sem = jnp.sum((blk >= offs[1, 1:3][None, :]).astype(jnp.int32), axis=1)

  g1 = _adj_mm(adj_matrix, t1.reshape(_B, _S, _D))
  t2 = _mm(g1.reshape(_T, _D), gcn2_W)
  shared = _adj_mm_ln(adj_matrix, t2.reshape(_B, _S, _D), hs3d, ln_g, ln_b)

  psyn_off = psyn + _TPAD
  sorted2 = _sc_scatter2(psem, psyn_off, hs2d, shared.reshape(_T, _D))

  len_sel = jnp.where(is_short, 0, 1).astype(jnp.int32)
  len_W2 = jnp.stack([lenS_W, lenL_W])
  len_b2 = jnp.stack([lenS_b, lenL_b])
  lenlog = _len_mm(len_sel, scl[:, 1:2], hs2d, len_W2, len_b2, cls_W, cls_b)

  w6 = jnp.concatenate([sem_W, syn_W], axis=0)
  b6 = jnp.concatenate([sem_b, syn_b], axis=0)
  be_all = jnp.concatenate([be_sem, be_syn + 3])
  alllog = _group_mm(be_all, sorted2, w6, b6, cls_W)

  logits = _sc_combine(psem, psyn_off, scl[:, 0], scl[:, 2],
                       alllog.reshape(4 * _TPAD), lenlog.reshape(2 * _T))
  return logits.reshape(_B, _S, 2)


# speedup vs baseline: 1.2095x; 1.2095x over previous
import functools

import jax
import jax.numpy as jnp
from jax import lax
from jax.experimental import pallas as pl
from jax.experimental.pallas import tpu as pltpu
from jax.experimental.pallas import tpu_sc as plsc

_B, _S, _D = 2, 2048, 1024
_T = _B * _S
_THRESHOLD = 128


_NW = 32
_GCH = 64


def _sc_gather(idx, table):
  t, d = idx.shape[0], table.shape[1]
  b_per_w = t // _NW
  n_ch = b_per_w // _GCH
  mesh = plsc.VectorSubcoreMesh(core_axis_name="c", subcore_axis_name="s")

  @functools.partial(
      pl.kernel,
      mesh=mesh,
      out_type=jax.ShapeDtypeStruct((t, d), jnp.float32),
      scratch_types=[
          pltpu.VMEM((_GCH,), jnp.int32),
          pltpu.VMEM((_GCH, d), jnp.float32),
          pltpu.SemaphoreType.DMA,
      ],
  )
  def k(idx_hbm, table_hbm, out_hbm, idx_v, rows_v, sem):
    wid = lax.axis_index("s") * 2 + lax.axis_index("c")
    base = wid * b_per_w
    for c in range(n_ch):
      off = base + c * _GCH
      pltpu.sync_copy(idx_hbm.at[pl.ds(off, _GCH)], idx_v)
      pltpu.async_copy(table_hbm.at[idx_v], rows_v, sem).wait()
      pltpu.sync_copy(rows_v, out_hbm.at[pl.ds(off, _GCH)])

  return k(idx, table)


_RTB = 512


def _router_body(short_ref, hs_ref, g1w_ref, w_ref, b_ref,
                 t1_ref, idx_ref, scl_ref):
  hs = hs_ref[...]
  t1_ref[...] = jnp.dot(hs, g1w_ref[...], preferred_element_type=jnp.float32)
  rl = jnp.dot(hs, w_ref[...], preferred_element_type=jnp.float32)
  rl = rl + b_ref[...]
  short = short_ref[0, 0, 0] != 0
  neg = jnp.float32(-1e9)
  col = lax.broadcasted_iota(jnp.int32, rl.shape, 1)
  rl = jnp.where(jnp.logical_and(col == 4, short), neg, rl)
  rl = jnp.where(jnp.logical_and(col == 3, jnp.logical_not(short)), neg, rl)
  m = jnp.max(rl, axis=-1, keepdims=True)
  e = jnp.exp(rl - m)
  p = e / jnp.sum(e, axis=-1, keepdims=True)
  syn_p = jnp.max(p[:, 0:3], axis=-1, keepdims=True)
  syn_i = jnp.argmax(p[:, 0:3], axis=-1, keepdims=True)
  len_p = jnp.max(p[:, 3:5], axis=-1, keepdims=True)
  sem_p = jnp.max(p[:, 5:8], axis=-1, keepdims=True)
  sem_i = jnp.argmax(p[:, 5:8], axis=-1, keepdims=True)
  tot = syn_p + len_p + sem_p
  scl = jnp.concatenate([syn_p, len_p, sem_p, tot], axis=-1) / tot
  scl_ref[...] = scl
  idx_ref[...] = jnp.concatenate([syn_i, sem_i], axis=-1).astype(jnp.int32)


def _mm1_router(hs2d, gcn1_W, router_W, router_b, is_short):
  grid = (_T // _RTB,)
  return pl.pallas_call(
      _router_body,
      grid=grid,
      in_specs=[
          pl.BlockSpec((1, 1, 1), lambda i: (i * _RTB // _S, 0, 0)),
          pl.BlockSpec((_RTB, _D), lambda i: (i, 0)),
          pl.BlockSpec((_D, _D), lambda i: (0, 0)),
          pl.BlockSpec((_D, 8), lambda i: (0, 0)),
          pl.BlockSpec((1, 8), lambda i: (0, 0)),
      ],
      out_specs=[
          pl.BlockSpec((_RTB, _D), lambda i: (i, 0)),
          pl.BlockSpec((_RTB, 2), lambda i: (i, 0)),
          pl.BlockSpec((_RTB, 4), lambda i: (i, 0)),
      ],
      out_shape=[
          jax.ShapeDtypeStruct((_T, _D), jnp.float32),
          jax.ShapeDtypeStruct((_T, 2), jnp.int32),
          jax.ShapeDtypeStruct((_T, 4), jnp.float32),
      ],
  )(is_short, hs2d, gcn1_W, router_W, router_b.reshape(1, 8))


_MMB = 512


def _mm_body(x_ref, w_ref, o_ref):
  o_ref[...] = jnp.dot(x_ref[...], w_ref[...],
                       preferred_element_type=jnp.float32)


def _mm(x, w):
  return pl.pallas_call(
      _mm_body,
      grid=(_T // _MMB,),
      in_specs=[
          pl.BlockSpec((_MMB, _D), lambda i: (i, 0)),
          pl.BlockSpec((_D, _D), lambda i: (0, 0)),
      ],
      out_specs=pl.BlockSpec((_MMB, _D), lambda i: (i, 0)),
      out_shape=jax.ShapeDtypeStruct((_T, _D), jnp.float32),
  )(x, w)


_AMB = 512


def _adj_body(adj_ref, sup_ref, o_ref):
  a = adj_ref[0]
  deg = jnp.maximum(jnp.sum(a, axis=-1, keepdims=True), 1e-9)
  acc = jnp.dot(a, sup_ref[0], preferred_element_type=jnp.float32)
  o_ref[...] = jnp.maximum(acc / deg, 0.0)[None]


def _adj_ln_body(adj_ref, sup_ref, hs_ref, g_ref, b_ref, o_ref):
  a = adj_ref[0]
  deg = jnp.maximum(jnp.sum(a, axis=-1, keepdims=True), 1e-9)
  acc = jnp.dot(a, sup_ref[0], preferred_element_type=jnp.float32)
  x = jnp.maximum(acc / deg, 0.0) + hs_ref[0]
  mu = jnp.mean(x, axis=-1, keepdims=True)
  xc = x - mu
  var = jnp.mean(xc * xc, axis=-1, keepdims=True)
  y = xc * lax.rsqrt(var + 1e-5) * g_ref[...] + b_ref[...]
  o_ref[...] = y[None]


def _adj_mm(adj, sup3d):
  return pl.pallas_call(
      _adj_body,
      grid=(_B, _S // _AMB),
      in_specs=[
          pl.BlockSpec((1, _AMB, _S), lambda b, i: (b, i, 0)),
          pl.BlockSpec((1, _S, _D), lambda b, i: (b, 0, 0)),
      ],
      out_specs=pl.BlockSpec((1, _AMB, _D), lambda b, i: (b, i, 0)),
      out_shape=jax.ShapeDtypeStruct((_B, _S, _D), jnp.float32),
  )(adj, sup3d)


def _adj_mm_ln(adj, sup3d, hs3d, ln_g, ln_b):
  return pl.pallas_call(
      _adj_ln_body,
      grid=(_B, _S // _AMB),
      in_specs=[
          pl.BlockSpec((1, _AMB, _S), lambda b, i: (b, i, 0)),
          pl.BlockSpec((1, _S, _D), lambda b, i: (b, 0, 0)),
          pl.BlockSpec((1, _AMB, _D), lambda b, i: (b, i, 0)),
          pl.BlockSpec((1, _D), lambda b, i: (0, 0)),
          pl.BlockSpec((1, _D), lambda b, i: (0, 0)),
      ],
      out_specs=pl.BlockSpec((1, _AMB, _D), lambda b, i: (b, i, 0)),
      out_shape=jax.ShapeDtypeStruct((_B, _S, _D), jnp.float32),
  )(adj, sup3d, hs3d, ln_g.reshape(1, _D), ln_b.reshape(1, _D))


_XB = 256


def _gelu(x):
  return 0.5 * x * (1.0 + lax.erf(x * 0.7071067811865476))


def _expert_body(sel_ref, idx_ref, scl_ref, sh_ref, hs_ref,
                 synw_ref, synb_ref, lensw_ref, lensb_ref,
                 lenlw_ref, lenlb_ref,
                 semw_ref, semb_ref, clsw_ref, clsb_ref, o_ref):
  sh = sh_ref[...].astype(jnp.bfloat16)
  h = hs_ref[...].astype(jnp.bfloat16)
  si = idx_ref[:, 0:1]
  mi = idx_ref[:, 1:2]
  fused = jnp.zeros((_XB, _D), jnp.float32)
  for i in range(3):
    eo = _gelu(jnp.dot(sh, synw_ref[i].astype(jnp.bfloat16),
                       preferred_element_type=jnp.float32)
               + synb_ref[i:i + 1, :])
    fused = fused + jnp.where(si == i, scl_ref[:, 0:1], 0.0) * eo
  for i in range(3):
    eo = _gelu(jnp.dot(h, semw_ref[i].astype(jnp.bfloat16),
                       preferred_element_type=jnp.float32)
               + semb_ref[i:i + 1, :])
    fused = fused + jnp.where(mi == i, scl_ref[:, 2:3], 0.0) * eo
  sel = sel_ref[pl.program_id(0) * _XB // _S]

  def _finish(lw_ref, lb_ref):
    lo = _gelu(jnp.dot(h, lw_ref[...].astype(jnp.bfloat16),
                       preferred_element_type=jnp.float32) + lb_ref[...])
    f2 = fused + scl_ref[:, 1:2] * lo
    o_ref[...] = (jnp.dot(f2, clsw_ref[...],
                          preferred_element_type=jnp.float32) + clsb_ref[...])

  @pl.when(sel == 0)
  def _():
    _finish(lensw_ref, lensb_ref)

  @pl.when(sel != 0)
  def _():
    _finish(lenlw_ref, lenlb_ref)


def _experts(len_sel, idx, scl, shared2d, hs2d, syn_W, syn_b,
             lenS_W, lenS_b, lenL_W, lenL_b, sem_W, sem_b, cls_W, cls_b):
  grid_spec = pltpu.PrefetchScalarGridSpec(
      num_scalar_prefetch=1,
      grid=(_T // _XB,),
      in_specs=[
          pl.BlockSpec((_XB, 2), lambda i, sel: (i, 0)),
          pl.BlockSpec((_XB, 4), lambda i, sel: (i, 0)),
          pl.BlockSpec((_XB, _D), lambda i, sel: (i, 0)),
          pl.BlockSpec((_XB, _D), lambda i, sel: (i, 0)),
          pl.BlockSpec((3, _D, _D), lambda i, sel: (0, 0, 0)),
          pl.BlockSpec((3, _D), lambda i, sel: (0, 0)),
          pl.BlockSpec((_D, _D), lambda i, sel: (0, 0)),
          pl.BlockSpec((1, _D), lambda i, sel: (0, 0)),
          pl.BlockSpec((_D, _D), lambda i, sel: (0, 0)),
          pl.BlockSpec((1, _D), lambda i, sel: (0, 0)),
          pl.BlockSpec((3, _D, _D), lambda i, sel: (0, 0, 0)),
          pl.BlockSpec((3, _D), lambda i, sel: (0, 0)),
          pl.BlockSpec((_D, 2), lambda i, sel: (0, 0)),
          pl.BlockSpec((1, 2), lambda i, sel: (0, 0)),
      ],
      out_specs=pl.BlockSpec((_XB, 2), lambda i, sel: (i, 0)),
  )
  return pl.pallas_call(
      _expert_body,
      grid_spec=grid_spec,
      out_shape=jax.ShapeDtypeStruct((_T, 2), jnp.float32),
  )(len_sel, idx, scl, shared2d, hs2d, syn_W, syn_b,
    lenS_W, lenS_b.reshape(1, _D), lenL_W, lenL_b.reshape(1, _D),
    sem_W, sem_b, cls_W, cls_b.reshape(1, 2))


def kernel(input_ids, attention_mask, seq_lengths, adj_matrix, emb, router_W,
           router_b, gcn1_W, gcn2_W, ln_g, ln_b, syn_W, syn_b, lenS_W, lenS_b,
           lenL_W, lenL_b, sem_W, sem_b, cls_W, cls_b):
  del attention_mask
  ids = input_ids.reshape(_T).astype(jnp.int32)
  hs2d = _sc_gather(ids, emb)
  hs3d = hs2d.reshape(_B, _S, _D)

  is_short = (seq_lengths <= _THRESHOLD)
  t1, idx, scl = _mm1_router(hs2d, gcn1_W, router_W, router_b,
                             is_short.astype(jnp.int32).reshape(_B, 1, 1))

  g1 = _adj_mm(adj_matrix, t1.reshape(_B, _S, _D))
  t2 = _mm(g1.reshape(_T, _D), gcn2_W)
  shared = _adj_mm_ln(adj_matrix, t2.reshape(_B, _S, _D), hs3d, ln_g, ln_b)

  len_sel = jnp.where(is_short, 0, 1).astype(jnp.int32)
  logits = _experts(len_sel, idx, scl, shared.reshape(_T, _D), hs2d,
                    syn_W, syn_b, lenS_W, lenS_b, lenL_W, lenL_b,
                    sem_W, sem_b, cls_W, cls_b)
  return logits.reshape(_B, _S, 2)


# speedup vs baseline: 1.2627x; 1.0440x over previous
import functools

import jax
import jax.numpy as jnp
from jax import lax
from jax.experimental import pallas as pl
from jax.experimental.pallas import tpu as pltpu
from jax.experimental.pallas import tpu_sc as plsc

_B, _S, _D = 2, 2048, 1024
_T = _B * _S
_THRESHOLD = 128


_NW = 32
_GCH = 64


def _sc_gather(idx, table):
  t, d = idx.shape[0], table.shape[1]
  b_per_w = t // _NW
  n_ch = b_per_w // _GCH
  mesh = plsc.VectorSubcoreMesh(core_axis_name="c", subcore_axis_name="s")

  @functools.partial(
      pl.kernel,
      mesh=mesh,
      out_type=jax.ShapeDtypeStruct((t, d), jnp.float32),
      scratch_types=[
          pltpu.VMEM((_GCH,), jnp.int32),
          pltpu.VMEM((_GCH, d), jnp.float32),
          pltpu.SemaphoreType.DMA,
      ],
  )
  def k(idx_hbm, table_hbm, out_hbm, idx_v, rows_v, sem):
    wid = lax.axis_index("s") * 2 + lax.axis_index("c")
    base = wid * b_per_w
    for c in range(n_ch):
      off = base + c * _GCH
      pltpu.sync_copy(idx_hbm.at[pl.ds(off, _GCH)], idx_v)
      pltpu.async_copy(table_hbm.at[idx_v], rows_v, sem).wait()
      pltpu.sync_copy(rows_v, out_hbm.at[pl.ds(off, _GCH)])

  return k(idx, table)


_RTB = 512


def _router_body(short_ref, hs_ref, g1w_ref, w_ref, b_ref,
                 t1_ref, idx_ref, scl_ref):
  hs = hs_ref[...]
  t1_ref[...] = jnp.dot(hs, g1w_ref[...], preferred_element_type=jnp.float32)
  rl = jnp.dot(hs, w_ref[...], preferred_element_type=jnp.float32)
  rl = rl + b_ref[...]
  short = short_ref[0, 0, 0] != 0
  neg = jnp.float32(-1e9)
  col = lax.broadcasted_iota(jnp.int32, rl.shape, 1)
  rl = jnp.where(jnp.logical_and(col == 4, short), neg, rl)
  rl = jnp.where(jnp.logical_and(col == 3, jnp.logical_not(short)), neg, rl)
  m = jnp.max(rl, axis=-1, keepdims=True)
  e = jnp.exp(rl - m)
  p = e / jnp.sum(e, axis=-1, keepdims=True)
  syn_p = jnp.max(p[:, 0:3], axis=-1, keepdims=True)
  syn_i = jnp.argmax(p[:, 0:3], axis=-1, keepdims=True)
  len_p = jnp.max(p[:, 3:5], axis=-1, keepdims=True)
  sem_p = jnp.max(p[:, 5:8], axis=-1, keepdims=True)
  sem_i = jnp.argmax(p[:, 5:8], axis=-1, keepdims=True)
  tot = syn_p + len_p + sem_p
  scl = jnp.concatenate([syn_p, len_p, sem_p, tot], axis=-1) / tot
  scl_ref[...] = scl
  idx_ref[...] = jnp.concatenate([syn_i, sem_i], axis=-1).astype(jnp.int32)


def _mm1_router(hs2d, gcn1_W, router_W, router_b, is_short):
  grid = (_T // _RTB,)
  return pl.pallas_call(
      _router_body,
      grid=grid,
      in_specs=[
          pl.BlockSpec((1, 1, 1), lambda i: (i * _RTB // _S, 0, 0)),
          pl.BlockSpec((_RTB, _D), lambda i: (i, 0)),
          pl.BlockSpec((_D, _D), lambda i: (0, 0)),
          pl.BlockSpec((_D, 8), lambda i: (0, 0)),
          pl.BlockSpec((1, 8), lambda i: (0, 0)),
      ],
      out_specs=[
          pl.BlockSpec((_RTB, _D), lambda i: (i, 0)),
          pl.BlockSpec((_RTB, 2), lambda i: (i, 0)),
          pl.BlockSpec((_RTB, 4), lambda i: (i, 0)),
      ],
      out_shape=[
          jax.ShapeDtypeStruct((_T, _D), jnp.float32),
          jax.ShapeDtypeStruct((_T, 2), jnp.int32),
          jax.ShapeDtypeStruct((_T, 4), jnp.float32),
      ],
  )(is_short, hs2d, gcn1_W, router_W, router_b.reshape(1, 8))


_MMB = 512


def _mm_body(x_ref, w_ref, o_ref):
  o_ref[...] = jnp.dot(x_ref[...], w_ref[...],
                       preferred_element_type=jnp.float32)


def _mm(x, w):
  return pl.pallas_call(
      _mm_body,
      grid=(_T // _MMB,),
      in_specs=[
          pl.BlockSpec((_MMB, _D), lambda i: (i, 0)),
          pl.BlockSpec((_D, _D), lambda i: (0, 0)),
      ],
      out_specs=pl.BlockSpec((_MMB, _D), lambda i: (i, 0)),
      out_shape=jax.ShapeDtypeStruct((_T, _D), jnp.float32),
  )(x, w)


_AMB = 512


def _adj_w_body(adj_ref, sup_ref, w2_ref, o_ref):
  a = adj_ref[0]
  deg = jnp.maximum(jnp.sum(a, axis=-1, keepdims=True), 1e-9)
  acc = jnp.dot(a, sup_ref[0], preferred_element_type=jnp.float32)
  g = jnp.maximum(acc / deg, 0.0)
  o_ref[...] = jnp.dot(g, w2_ref[...], preferred_element_type=jnp.float32)[None]


def _adj_ln_body(adj_ref, sup_ref, hs_ref, g_ref, b_ref, o_ref):
  a = adj_ref[0]
  deg = jnp.maximum(jnp.sum(a, axis=-1, keepdims=True), 1e-9)
  acc = jnp.dot(a, sup_ref[0], preferred_element_type=jnp.float32)
  x = jnp.maximum(acc / deg, 0.0) + hs_ref[0]
  mu = jnp.mean(x, axis=-1, keepdims=True)
  xc = x - mu
  var = jnp.mean(xc * xc, axis=-1, keepdims=True)
  y = xc * lax.rsqrt(var + 1e-5) * g_ref[...] + b_ref[...]
  o_ref[...] = y[None]


def _adj_mm_w(adj, sup3d, w2):
  return pl.pallas_call(
      _adj_w_body,
      grid=(_B, _S // _AMB),
      in_specs=[
          pl.BlockSpec((1, _AMB, _S), lambda b, i: (b, i, 0)),
          pl.BlockSpec((1, _S, _D), lambda b, i: (b, 0, 0)),
          pl.BlockSpec((_D, _D), lambda b, i: (0, 0)),
      ],
      out_specs=pl.BlockSpec((1, _AMB, _D), lambda b, i: (b, i, 0)),
      out_shape=jax.ShapeDtypeStruct((_B, _S, _D), jnp.float32),
  )(adj, sup3d, w2)


def _adj_mm_ln(adj, sup3d, hs3d, ln_g, ln_b):
  return pl.pallas_call(
      _adj_ln_body,
      grid=(_B, _S // _AMB),
      in_specs=[
          pl.BlockSpec((1, _AMB, _S), lambda b, i: (b, i, 0)),
          pl.BlockSpec((1, _S, _D), lambda b, i: (b, 0, 0)),
          pl.BlockSpec((1, _AMB, _D), lambda b, i: (b, i, 0)),
          pl.BlockSpec((1, _D), lambda b, i: (0, 0)),
          pl.BlockSpec((1, _D), lambda b, i: (0, 0)),
      ],
      out_specs=pl.BlockSpec((1, _AMB, _D), lambda b, i: (b, i, 0)),
      out_shape=jax.ShapeDtypeStruct((_B, _S, _D), jnp.float32),
  )(adj, sup3d, hs3d, ln_g.reshape(1, _D), ln_b.reshape(1, _D))


_XB = 256


def _gelu(x):
  return 0.5 * x * (1.0 + lax.erf(x * 0.7071067811865476))


def _expert_body(sel_ref, idx_ref, scl_ref, sh_ref, hs_ref,
                 synw_ref, synb_ref, lensw_ref, lensb_ref,
                 lenlw_ref, lenlb_ref,
                 semw_ref, semb_ref, clsw_ref, clsb_ref, o_ref):
  sh = sh_ref[...].astype(jnp.bfloat16)
  h = hs_ref[...].astype(jnp.bfloat16)
  si = idx_ref[:, 0:1]
  mi = idx_ref[:, 1:2]
  fused = jnp.zeros((_XB, _D), jnp.float32)
  for i in range(3):
    eo = _gelu(jnp.dot(sh, synw_ref[i].astype(jnp.bfloat16),
                       preferred_element_type=jnp.float32)
               + synb_ref[i:i + 1, :])
    fused = fused + jnp.where(si == i, scl_ref[:, 0:1], 0.0) * eo
  for i in range(3):
    eo = _gelu(jnp.dot(h, semw_ref[i].astype(jnp.bfloat16),
                       preferred_element_type=jnp.float32)
               + semb_ref[i:i + 1, :])
    fused = fused + jnp.where(mi == i, scl_ref[:, 2:3], 0.0) * eo
  sel = sel_ref[pl.program_id(0) * _XB // _S]

  def _finish(lw_ref, lb_ref):
    lo = _gelu(jnp.dot(h, lw_ref[...].astype(jnp.bfloat16),
                       preferred_element_type=jnp.float32) + lb_ref[...])
    f2 = fused + scl_ref[:, 1:2] * lo
    o_ref[...] = (jnp.dot(f2, clsw_ref[...],
                          preferred_element_type=jnp.float32) + clsb_ref[...])

  @pl.when(sel == 0)
  def _():
    _finish(lensw_ref, lensb_ref)

  @pl.when(sel != 0)
  def _():
    _finish(lenlw_ref, lenlb_ref)


def _experts(len_sel, idx, scl, shared2d, hs2d, syn_W, syn_b,
             lenS_W, lenS_b, lenL_W, lenL_b, sem_W, sem_b, cls_W, cls_b):
  grid_spec = pltpu.PrefetchScalarGridSpec(
      num_scalar_prefetch=1,
      grid=(_T // _XB,),
      in_specs=[
          pl.BlockSpec((_XB, 2), lambda i, sel: (i, 0)),
          pl.BlockSpec((_XB, 4), lambda i, sel: (i, 0)),
          pl.BlockSpec((_XB, _D), lambda i, sel: (i, 0)),
          pl.BlockSpec((_XB, _D), lambda i, sel: (i, 0)),
          pl.BlockSpec((3, _D, _D), lambda i, sel: (0, 0, 0)),
          pl.BlockSpec((3, _D), lambda i, sel: (0, 0)),
          pl.BlockSpec((_D, _D), lambda i, sel: (0, 0)),
          pl.BlockSpec((1, _D), lambda i, sel: (0, 0)),
          pl.BlockSpec((_D, _D), lambda i, sel: (0, 0)),
          pl.BlockSpec((1, _D), lambda i, sel: (0, 0)),
          pl.BlockSpec((3, _D, _D), lambda i, sel: (0, 0, 0)),
          pl.BlockSpec((3, _D), lambda i, sel: (0, 0)),
          pl.BlockSpec((_D, 2), lambda i, sel: (0, 0)),
          pl.BlockSpec((1, 2), lambda i, sel: (0, 0)),
      ],
      out_specs=pl.BlockSpec((_XB, 2), lambda i, sel: (i, 0)),
  )
  return pl.pallas_call(
      _expert_body,
      grid_spec=grid_spec,
      out_shape=jax.ShapeDtypeStruct((_T, 2), jnp.float32),
  )(len_sel, idx, scl, shared2d, hs2d, syn_W, syn_b,
    lenS_W, lenS_b.reshape(1, _D), lenL_W, lenL_b.reshape(1, _D),
    sem_W, sem_b, cls_W, cls_b.reshape(1, 2))


def kernel(input_ids, attention_mask, seq_lengths, adj_matrix, emb, router_W,
           router_b, gcn1_W, gcn2_W, ln_g, ln_b, syn_W, syn_b, lenS_W, lenS_b,
           lenL_W, lenL_b, sem_W, sem_b, cls_W, cls_b):
  del attention_mask
  ids = input_ids.reshape(_T).astype(jnp.int32)
  hs2d = _sc_gather(ids, emb)
  hs3d = hs2d.reshape(_B, _S, _D)

  is_short = (seq_lengths <= _THRESHOLD)
  t1, idx, scl = _mm1_router(hs2d, gcn1_W, router_W, router_b,
                             is_short.astype(jnp.int32).reshape(_B, 1, 1))

  t2 = _adj_mm_w(adj_matrix, t1.reshape(_B, _S, _D), gcn2_W)
  shared = _adj_mm_ln(adj_matrix, t2, hs3d, ln_g, ln_b)

  len_sel = jnp.where(is_short, 0, 1).astype(jnp.int32)
  logits = _experts(len_sel, idx, scl, shared.reshape(_T, _D), hs2d,
                    syn_W, syn_b, lenS_W, lenS_b, lenL_W, lenL_b,
                    sem_W, sem_b, cls_W, cls_b)
  return logits.reshape(_B, _S, 2)


# speedup vs baseline: 1.3324x; 1.0552x over previous
import functools

import jax
import jax.numpy as jnp
from jax import lax
from jax.experimental import pallas as pl
from jax.experimental.pallas import tpu as pltpu
from jax.experimental.pallas import tpu_sc as plsc

_B, _S, _D = 2, 2048, 1024
_T = _B * _S
_THRESHOLD = 128


_NW = 32
_GCH = 64


def _sc_gather(idx, table):
  t, d = idx.shape[0], table.shape[1]
  b_per_w = t // _NW
  n_ch = b_per_w // _GCH
  mesh = plsc.VectorSubcoreMesh(core_axis_name="c", subcore_axis_name="s")

  @functools.partial(
      pl.kernel,
      mesh=mesh,
      out_type=jax.ShapeDtypeStruct((t, d), jnp.float32),
      scratch_types=[
          pltpu.VMEM((_GCH,), jnp.int32),
          pltpu.VMEM((_GCH, d), jnp.float32),
          pltpu.SemaphoreType.DMA,
      ],
  )
  def k(idx_hbm, table_hbm, out_hbm, idx_v, rows_v, sem):
    wid = lax.axis_index("s") * 2 + lax.axis_index("c")
    base = wid * b_per_w
    for c in range(n_ch):
      off = base + c * _GCH
      pltpu.sync_copy(idx_hbm.at[pl.ds(off, _GCH)], idx_v)
      pltpu.async_copy(table_hbm.at[idx_v], rows_v, sem).wait()
      pltpu.sync_copy(rows_v, out_hbm.at[pl.ds(off, _GCH)])

  return k(idx, table)


_RTB = 512


def _router_body(short_ref, hs_ref, g1w_ref, w_ref, b_ref,
                 t1_ref, idx_ref, scl_ref):
  hs = hs_ref[...]
  t1_ref[...] = jnp.dot(hs, g1w_ref[...], preferred_element_type=jnp.float32)
  rl = jnp.dot(hs, w_ref[...], preferred_element_type=jnp.float32)
  rl = rl + b_ref[...]
  short = short_ref[0, 0, 0] != 0
  neg = jnp.float32(-1e9)
  col = lax.broadcasted_iota(jnp.int32, rl.shape, 1)
  rl = jnp.where(jnp.logical_and(col == 4, short), neg, rl)
  rl = jnp.where(jnp.logical_and(col == 3, jnp.logical_not(short)), neg, rl)
  m = jnp.max(rl, axis=-1, keepdims=True)
  e = jnp.exp(rl - m)
  p = e / jnp.sum(e, axis=-1, keepdims=True)
  syn_p = jnp.max(p[:, 0:3], axis=-1, keepdims=True)
  syn_i = jnp.argmax(p[:, 0:3], axis=-1, keepdims=True)
  len_p = jnp.max(p[:, 3:5], axis=-1, keepdims=True)
  sem_p = jnp.max(p[:, 5:8], axis=-1, keepdims=True)
  sem_i = jnp.argmax(p[:, 5:8], axis=-1, keepdims=True)
  tot = syn_p + len_p + sem_p
  scl = jnp.concatenate([syn_p, len_p, sem_p, tot], axis=-1) / tot
  scl_ref[...] = scl
  idx_ref[...] = jnp.concatenate([syn_i, sem_i], axis=-1).astype(jnp.int32)


def _mm1_router(hs2d, gcn1_W, router_W, router_b, is_short):
  grid = (_T // _RTB,)
  return pl.pallas_call(
      _router_body,
      grid=grid,
      in_specs=[
          pl.BlockSpec((1, 1, 1), lambda i: (i * _RTB // _S, 0, 0)),
          pl.BlockSpec((_RTB, _D), lambda i: (i, 0)),
          pl.BlockSpec((_D, _D), lambda i: (0, 0)),
          pl.BlockSpec((_D, 8), lambda i: (0, 0)),
          pl.BlockSpec((1, 8), lambda i: (0, 0)),
      ],
      out_specs=[
          pl.BlockSpec((_RTB, _D), lambda i: (i, 0)),
          pl.BlockSpec((_RTB, 2), lambda i: (i, 0)),
          pl.BlockSpec((_RTB, 4), lambda i: (i, 0)),
      ],
      out_shape=[
          jax.ShapeDtypeStruct((_T, _D), jnp.float32),
          jax.ShapeDtypeStruct((_T, 2), jnp.int32),
          jax.ShapeDtypeStruct((_T, 4), jnp.float32),
      ],
  )(is_short, hs2d, gcn1_W, router_W, router_b.reshape(1, 8))


_MMB = 512


def _mm_body(x_ref, w_ref, o_ref):
  o_ref[...] = jnp.dot(x_ref[...], w_ref[...],
                       preferred_element_type=jnp.float32)


def _mm(x, w):
  return pl.pallas_call(
      _mm_body,
      grid=(_T // _MMB,),
      in_specs=[
          pl.BlockSpec((_MMB, _D), lambda i: (i, 0)),
          pl.BlockSpec((_D, _D), lambda i: (0, 0)),
      ],
      out_specs=pl.BlockSpec((_MMB, _D), lambda i: (i, 0)),
      out_shape=jax.ShapeDtypeStruct((_T, _D), jnp.float32),
  )(x, w)


_AMB = 512


def _adj_w_body(adj_ref, sup_ref, w2_ref, o_ref):
  a = adj_ref[0]
  deg = jnp.maximum(jnp.sum(a, axis=-1, keepdims=True), 1e-9)
  acc = jnp.dot(a, sup_ref[0], preferred_element_type=jnp.float32)
  g = jnp.maximum(acc / deg, 0.0)
  o_ref[...] = jnp.dot(g, w2_ref[...], preferred_element_type=jnp.float32)[None]


def _adj_ln_body(adj_ref, sup_ref, hs_ref, g_ref, b_ref, o_ref):
  a = adj_ref[0]
  deg = jnp.maximum(jnp.sum(a, axis=-1, keepdims=True), 1e-9)
  acc = jnp.dot(a, sup_ref[0], preferred_element_type=jnp.float32)
  x = jnp.maximum(acc / deg, 0.0) + hs_ref[0]
  mu = jnp.mean(x, axis=-1, keepdims=True)
  xc = x - mu
  var = jnp.mean(xc * xc, axis=-1, keepdims=True)
  y = xc * lax.rsqrt(var + 1e-5) * g_ref[...] + b_ref[...]
  o_ref[...] = y[None]


def _adj_mm_w(adj, sup3d, w2):
  return pl.pallas_call(
      _adj_w_body,
      grid=(_B, _S // _AMB),
      in_specs=[
          pl.BlockSpec((1, _AMB, _S), lambda b, i: (b, i, 0)),
          pl.BlockSpec((1, _S, _D), lambda b, i: (b, 0, 0)),
          pl.BlockSpec((_D, _D), lambda b, i: (0, 0)),
      ],
      out_specs=pl.BlockSpec((1, _AMB, _D), lambda b, i: (b, i, 0)),
      out_shape=jax.ShapeDtypeStruct((_B, _S, _D), jnp.float32),
  )(adj, sup3d, w2)


def _adj_mm_ln(adj, sup3d, hs3d, ln_g, ln_b):
  return pl.pallas_call(
      _adj_ln_body,
      grid=(_B, _S // _AMB),
      in_specs=[
          pl.BlockSpec((1, _AMB, _S), lambda b, i: (b, i, 0)),
          pl.BlockSpec((1, _S, _D), lambda b, i: (b, 0, 0)),
          pl.BlockSpec((1, _AMB, _D), lambda b, i: (b, i, 0)),
          pl.BlockSpec((1, _D), lambda b, i: (0, 0)),
          pl.BlockSpec((1, _D), lambda b, i: (0, 0)),
      ],
      out_specs=pl.BlockSpec((1, _AMB, _D), lambda b, i: (b, i, 0)),
      out_shape=jax.ShapeDtypeStruct((_B, _S, _D), jnp.float32),
  )(adj, sup3d, hs3d, ln_g.reshape(1, _D), ln_b.reshape(1, _D))


_XB = 512


def _gelu(x):
  return 0.5 * x * (1.0 + lax.erf(x * 0.7071067811865476))


def _expert_body(sel_ref, idx_ref, scl_ref, sh_ref, hs_ref,
                 synw_ref, synb_ref, lensw_ref, lensb_ref,
                 lenlw_ref, lenlb_ref,
                 semw_ref, semb_ref, clsw_ref, clsb_ref, o_ref):
  sh = sh_ref[...].astype(jnp.bfloat16)
  h = hs_ref[...].astype(jnp.bfloat16)
  si = idx_ref[:, 0:1]
  mi = idx_ref[:, 1:2]
  fused = jnp.zeros((_XB, _D), jnp.float32)
  for i in range(3):
    eo = _gelu(jnp.dot(sh, synw_ref[i].astype(jnp.bfloat16),
                       preferred_element_type=jnp.float32)
               + synb_ref[i:i + 1, :])
    fused = fused + jnp.where(si == i, scl_ref[:, 0:1], 0.0) * eo
  for i in range(3):
    eo = _gelu(jnp.dot(h, semw_ref[i].astype(jnp.bfloat16),
                       preferred_element_type=jnp.float32)
               + semb_ref[i:i + 1, :])
    fused = fused + jnp.where(mi == i, scl_ref[:, 2:3], 0.0) * eo
  sel = sel_ref[pl.program_id(0) * _XB // _S]

  def _finish(lw_ref, lb_ref):
    lo = _gelu(jnp.dot(h, lw_ref[...].astype(jnp.bfloat16),
                       preferred_element_type=jnp.float32) + lb_ref[...])
    f2 = fused + scl_ref[:, 1:2] * lo
    o_ref[...] = (jnp.dot(f2, clsw_ref[...],
                          preferred_element_type=jnp.float32) + clsb_ref[...])

  @pl.when(sel == 0)
  def _():
    _finish(lensw_ref, lensb_ref)

  @pl.when(sel != 0)
  def _():
    _finish(lenlw_ref, lenlb_ref)


def _experts(len_sel, idx, scl, shared2d, hs2d, syn_W, syn_b,
             lenS_W, lenS_b, lenL_W, lenL_b, sem_W, sem_b, cls_W, cls_b):
  grid_spec = pltpu.PrefetchScalarGridSpec(
      num_scalar_prefetch=1,
      grid=(_T // _XB,),
      in_specs=[
          pl.BlockSpec((_XB, 2), lambda i, sel: (i, 0)),
          pl.BlockSpec((_XB, 4), lambda i, sel: (i, 0)),
          pl.BlockSpec((_XB, _D), lambda i, sel: (i, 0)),
          pl.BlockSpec((_XB, _D), lambda i, sel: (i, 0)),
          pl.BlockSpec((3, _D, _D), lambda i, sel: (0, 0, 0)),
          pl.BlockSpec((3, _D), lambda i, sel: (0, 0)),
          pl.BlockSpec((_D, _D), lambda i, sel: (0, 0)),
          pl.BlockSpec((1, _D), lambda i, sel: (0, 0)),
          pl.BlockSpec((_D, _D), lambda i, sel: (0, 0)),
          pl.BlockSpec((1, _D), lambda i, sel: (0, 0)),
          pl.BlockSpec((3, _D, _D), lambda i, sel: (0, 0, 0)),
          pl.BlockSpec((3, _D), lambda i, sel: (0, 0)),
          pl.BlockSpec((_D, 2), lambda i, sel: (0, 0)),
          pl.BlockSpec((1, 2), lambda i, sel: (0, 0)),
      ],
      out_specs=pl.BlockSpec((_XB, 2), lambda i, sel: (i, 0)),
  )
  return pl.pallas_call(
      _expert_body,
      grid_spec=grid_spec,
      out_shape=jax.ShapeDtypeStruct((_T, 2), jnp.float32),
  )(len_sel, idx, scl, shared2d, hs2d, syn_W, syn_b,
    lenS_W, lenS_b.reshape(1, _D), lenL_W, lenL_b.reshape(1, _D),
    sem_W, sem_b, cls_W, cls_b.reshape(1, 2))


def kernel(input_ids, attention_mask, seq_lengths, adj_matrix, emb, router_W,
           router_b, gcn1_W, gcn2_W, ln_g, ln_b, syn_W, syn_b, lenS_W, lenS_b,
           lenL_W, lenL_b, sem_W, sem_b, cls_W, cls_b):
  del attention_mask
  ids = input_ids.reshape(_T).astype(jnp.int32)
  hs2d = _sc_gather(ids, emb)
  hs3d = hs2d.reshape(_B, _S, _D)

  is_short = (seq_lengths <= _THRESHOLD)
  t1, idx, scl = _mm1_router(hs2d, gcn1_W, router_W, router_b,
                             is_short.astype(jnp.int32).reshape(_B, 1, 1))

  t2 = _adj_mm_w(adj_matrix, t1.reshape(_B, _S, _D), gcn2_W)
  shared = _adj_mm_ln(adj_matrix, t2, hs3d, ln_g, ln_b)

  len_sel = jnp.where(is_short, 0, 1).astype(jnp.int32)
  logits = _experts(len_sel, idx, scl, shared.reshape(_T, _D), hs2d,
                    syn_W, syn_b, lenS_W, lenS_b, lenL_W, lenL_b,
                    sem_W, sem_b, cls_W, cls_b)
  return logits.reshape(_B, _S, 2)


# speedup vs baseline: 1.3394x; 1.0052x over previous
import functools

import jax
import jax.numpy as jnp
from jax import lax
from jax.experimental import pallas as pl
from jax.experimental.pallas import tpu as pltpu
from jax.experimental.pallas import tpu_sc as plsc

_B, _S, _D = 2, 2048, 1024
_T = _B * _S
_THRESHOLD = 128


_NW = 32
_GCH = 64


def _sc_gather(idx, table):
  t, d = idx.shape[0], table.shape[1]
  b_per_w = t // _NW
  n_ch = b_per_w // _GCH
  mesh = plsc.VectorSubcoreMesh(core_axis_name="c", subcore_axis_name="s")

  @functools.partial(
      pl.kernel,
      mesh=mesh,
      out_type=jax.ShapeDtypeStruct((t, d), jnp.float32),
      scratch_types=[
          pltpu.VMEM((_GCH,), jnp.int32),
          pltpu.VMEM((_GCH, d), jnp.float32),
          pltpu.SemaphoreType.DMA,
      ],
  )
  def k(idx_hbm, table_hbm, out_hbm, idx_v, rows_v, sem):
    wid = lax.axis_index("s") * 2 + lax.axis_index("c")
    base = wid * b_per_w
    for c in range(n_ch):
      off = base + c * _GCH
      pltpu.sync_copy(idx_hbm.at[pl.ds(off, _GCH)], idx_v)
      pltpu.async_copy(table_hbm.at[idx_v], rows_v, sem).wait()
      pltpu.sync_copy(rows_v, out_hbm.at[pl.ds(off, _GCH)])

  return k(idx, table)


_RTB = 1024


def _router_body(short_ref, hs_ref, g1w_ref, w_ref, b_ref,
                 t1_ref, idx_ref, scl_ref):
  hs = hs_ref[...]
  t1_ref[...] = jnp.dot(hs, g1w_ref[...], preferred_element_type=jnp.float32)
  rl = jnp.dot(hs, w_ref[...], preferred_element_type=jnp.float32)
  rl = rl + b_ref[...]
  short = short_ref[0, 0, 0] != 0
  neg = jnp.float32(-1e9)
  col = lax.broadcasted_iota(jnp.int32, rl.shape, 1)
  rl = jnp.where(jnp.logical_and(col == 4, short), neg, rl)
  rl = jnp.where(jnp.logical_and(col == 3, jnp.logical_not(short)), neg, rl)
  m = jnp.max(rl, axis=-1, keepdims=True)
  e = jnp.exp(rl - m)
  p = e / jnp.sum(e, axis=-1, keepdims=True)
  syn_p = jnp.max(p[:, 0:3], axis=-1, keepdims=True)
  syn_i = jnp.argmax(p[:, 0:3], axis=-1, keepdims=True)
  len_p = jnp.max(p[:, 3:5], axis=-1, keepdims=True)
  sem_p = jnp.max(p[:, 5:8], axis=-1, keepdims=True)
  sem_i = jnp.argmax(p[:, 5:8], axis=-1, keepdims=True)
  tot = syn_p + len_p + sem_p
  scl = jnp.concatenate([syn_p, len_p, sem_p, tot], axis=-1) / tot
  scl_ref[...] = scl
  idx_ref[...] = jnp.concatenate([syn_i, sem_i], axis=-1).astype(jnp.int32)


def _mm1_router(hs2d, gcn1_W, router_W, router_b, is_short):
  grid = (_T // _RTB,)
  return pl.pallas_call(
      _router_body,
      grid=grid,
      in_specs=[
          pl.BlockSpec((1, 1, 1), lambda i: (i * _RTB // _S, 0, 0)),
          pl.BlockSpec((_RTB, _D), lambda i: (i, 0)),
          pl.BlockSpec((_D, _D), lambda i: (0, 0)),
          pl.BlockSpec((_D, 8), lambda i: (0, 0)),
          pl.BlockSpec((1, 8), lambda i: (0, 0)),
      ],
      out_specs=[
          pl.BlockSpec((_RTB, _D), lambda i: (i, 0)),
          pl.BlockSpec((_RTB, 2), lambda i: (i, 0)),
          pl.BlockSpec((_RTB, 4), lambda i: (i, 0)),
      ],
      out_shape=[
          jax.ShapeDtypeStruct((_T, _D), jnp.float32),
          jax.ShapeDtypeStruct((_T, 2), jnp.int32),
          jax.ShapeDtypeStruct((_T, 4), jnp.float32),
      ],
  )(is_short, hs2d, gcn1_W, router_W, router_b.reshape(1, 8))


_MMB = 512


def _mm_body(x_ref, w_ref, o_ref):
  o_ref[...] = jnp.dot(x_ref[...], w_ref[...],
                       preferred_element_type=jnp.float32)


def _mm(x, w):
  return pl.pallas_call(
      _mm_body,
      grid=(_T // _MMB,),
      in_specs=[
          pl.BlockSpec((_MMB, _D), lambda i: (i, 0)),
          pl.BlockSpec((_D, _D), lambda i: (0, 0)),
      ],
      out_specs=pl.BlockSpec((_MMB, _D), lambda i: (i, 0)),
      out_shape=jax.ShapeDtypeStruct((_T, _D), jnp.float32),
  )(x, w)


_AMB = 1024


def _adj_w_body(adj_ref, sup_ref, w2_ref, o_ref):
  a = adj_ref[0]
  deg = jnp.maximum(jnp.sum(a, axis=-1, keepdims=True), 1e-9)
  acc = jnp.dot(a, sup_ref[0], preferred_element_type=jnp.float32)
  g = jnp.maximum(acc / deg, 0.0)
  o_ref[...] = jnp.dot(g, w2_ref[...], preferred_element_type=jnp.float32)[None]


def _adj_ln_body(adj_ref, sup_ref, hs_ref, g_ref, b_ref, o_ref):
  a = adj_ref[0]
  deg = jnp.maximum(jnp.sum(a, axis=-1, keepdims=True), 1e-9)
  acc = jnp.dot(a, sup_ref[0], preferred_element_type=jnp.float32)
  x = jnp.maximum(acc / deg, 0.0) + hs_ref[0]
  mu = jnp.mean(x, axis=-1, keepdims=True)
  xc = x - mu
  var = jnp.mean(xc * xc, axis=-1, keepdims=True)
  y = xc * lax.rsqrt(var + 1e-5) * g_ref[...] + b_ref[...]
  o_ref[...] = y[None]


def _adj_mm_w(adj, sup3d, w2):
  return pl.pallas_call(
      _adj_w_body,
      grid=(_B, _S // _AMB),
      in_specs=[
          pl.BlockSpec((1, _AMB, _S), lambda b, i: (b, i, 0)),
          pl.BlockSpec((1, _S, _D), lambda b, i: (b, 0, 0)),
          pl.BlockSpec((_D, _D), lambda b, i: (0, 0)),
      ],
      out_specs=pl.BlockSpec((1, _AMB, _D), lambda b, i: (b, i, 0)),
      out_shape=jax.ShapeDtypeStruct((_B, _S, _D), jnp.float32),
  )(adj, sup3d, w2)


def _adj_mm_ln(adj, sup3d, hs3d, ln_g, ln_b):
  return pl.pallas_call(
      _adj_ln_body,
      grid=(_B, _S // _AMB),
      in_specs=[
          pl.BlockSpec((1, _AMB, _S), lambda b, i: (b, i, 0)),
          pl.BlockSpec((1, _S, _D), lambda b, i: (b, 0, 0)),
          pl.BlockSpec((1, _AMB, _D), lambda b, i: (b, i, 0)),
          pl.BlockSpec((1, _D), lambda b, i: (0, 0)),
          pl.BlockSpec((1, _D), lambda b, i: (0, 0)),
      ],
      out_specs=pl.BlockSpec((1, _AMB, _D), lambda b, i: (b, i, 0)),
      out_shape=jax.ShapeDtypeStruct((_B, _S, _D), jnp.float32),
  )(adj, sup3d, hs3d, ln_g.reshape(1, _D), ln_b.reshape(1, _D))


_XB = 512


def _gelu(x):
  return 0.5 * x * (1.0 + lax.erf(x * 0.7071067811865476))


def _expert_body(sel_ref, idx_ref, scl_ref, sh_ref, hs_ref,
                 synw_ref, synb_ref, lensw_ref, lensb_ref,
                 lenlw_ref, lenlb_ref,
                 semw_ref, semb_ref, clsw_ref, clsb_ref, o_ref):
  sh = sh_ref[...].astype(jnp.bfloat16)
  h = hs_ref[...].astype(jnp.bfloat16)
  si = idx_ref[:, 0:1]
  mi = idx_ref[:, 1:2]
  fused = jnp.zeros((_XB, _D), jnp.float32)
  for i in range(3):
    eo = _gelu(jnp.dot(sh, synw_ref[i].astype(jnp.bfloat16),
                       preferred_element_type=jnp.float32)
               + synb_ref[i:i + 1, :])
    fused = fused + jnp.where(si == i, scl_ref[:, 0:1], 0.0) * eo
  for i in range(3):
    eo = _gelu(jnp.dot(h, semw_ref[i].astype(jnp.bfloat16),
                       preferred_element_type=jnp.float32)
               + semb_ref[i:i + 1, :])
    fused = fused + jnp.where(mi == i, scl_ref[:, 2:3], 0.0) * eo
  sel = sel_ref[pl.program_id(0) * _XB // _S]

  def _finish(lw_ref, lb_ref):
    lo = _gelu(jnp.dot(h, lw_ref[...].astype(jnp.bfloat16),
                       preferred_element_type=jnp.float32) + lb_ref[...])
    f2 = fused + scl_ref[:, 1:2] * lo
    o_ref[...] = (jnp.dot(f2, clsw_ref[...],
                          preferred_element_type=jnp.float32) + clsb_ref[...])

  @pl.when(sel == 0)
  def _():
    _finish(lensw_ref, lensb_ref)

  @pl.when(sel != 0)
  def _():
    _finish(lenlw_ref, lenlb_ref)


def _experts(len_sel, idx, scl, shared2d, hs2d, syn_W, syn_b,
             lenS_W, lenS_b, lenL_W, lenL_b, sem_W, sem_b, cls_W, cls_b):
  grid_spec = pltpu.PrefetchScalarGridSpec(
      num_scalar_prefetch=1,
      grid=(_T // _XB,),
      in_specs=[
          pl.BlockSpec((_XB, 2), lambda i, sel: (i, 0)),
          pl.BlockSpec((_XB, 4), lambda i, sel: (i, 0)),
          pl.BlockSpec((_XB, _D), lambda i, sel: (i, 0)),
          pl.BlockSpec((_XB, _D), lambda i, sel: (i, 0)),
          pl.BlockSpec((3, _D, _D), lambda i, sel: (0, 0, 0)),
          pl.BlockSpec((3, _D), lambda i, sel: (0, 0)),
          pl.BlockSpec((_D, _D), lambda i, sel: (0, 0)),
          pl.BlockSpec((1, _D), lambda i, sel: (0, 0)),
          pl.BlockSpec((_D, _D), lambda i, sel: (0, 0)),
          pl.BlockSpec((1, _D), lambda i, sel: (0, 0)),
          pl.BlockSpec((3, _D, _D), lambda i, sel: (0, 0, 0)),
          pl.BlockSpec((3, _D), lambda i, sel: (0, 0)),
          pl.BlockSpec((_D, 2), lambda i, sel: (0, 0)),
          pl.BlockSpec((1, 2), lambda i, sel: (0, 0)),
      ],
      out_specs=pl.BlockSpec((_XB, 2), lambda i, sel: (i, 0)),
  )
  return pl.pallas_call(
      _expert_body,
      grid_spec=grid_spec,
      out_shape=jax.ShapeDtypeStruct((_T, 2), jnp.float32),
  )(len_sel, idx, scl, shared2d, hs2d, syn_W, syn_b,
    lenS_W, lenS_b.reshape(1, _D), lenL_W, lenL_b.reshape(1, _D),
    sem_W, sem_b, cls_W, cls_b.reshape(1, 2))


def kernel(input_ids, attention_mask, seq_lengths, adj_matrix, emb, router_W,
           router_b, gcn1_W, gcn2_W, ln_g, ln_b, syn_W, syn_b, lenS_W, lenS_b,
           lenL_W, lenL_b, sem_W, sem_b, cls_W, cls_b):
  del attention_mask
  ids = input_ids.reshape(_T).astype(jnp.int32)
  hs2d = _sc_gather(ids, emb)
  hs3d = hs2d.reshape(_B, _S, _D)

  is_short = (seq_lengths <= _THRESHOLD)
  t1, idx, scl = _mm1_router(hs2d, gcn1_W, router_W, router_b,
                             is_short.astype(jnp.int32).reshape(_B, 1, 1))

  t2 = _adj_mm_w(adj_matrix, t1.reshape(_B, _S, _D), gcn2_W)
  shared = _adj_mm_ln(adj_matrix, t2, hs3d, ln_g, ln_b)

  len_sel = jnp.where(is_short, 0, 1).astype(jnp.int32)
  logits = _experts(len_sel, idx, scl, shared.reshape(_T, _D), hs2d,
                    syn_W, syn_b, lenS_W, lenS_b, lenL_W, lenL_b,
                    sem_W, sem_b, cls_W, cls_b)
  return logits.reshape(_B, _S, 2)
